# Initial kernel scaffold; baseline (speedup 1.0000x reference)
import jax
import jax.numpy as jnp
from jax.experimental import pallas as pl


def kernel(emb, table):
    raise NotImplementedError("write your pallas kernel here")


# speedup vs baseline: 3.7613x; 3.7613x over previous
import functools

import jax
import jax.numpy as jnp
from jax import lax
from jax.experimental import pallas as pl
from jax.experimental.pallas import tpu as pltpu
from jax.experimental.pallas import tpu_sc as plsc

NUM_EMBEDDINGS = 100000
EMBED_DIM = 64
B_ROWS = 4096
B_COLS = 100
TOTAL = B_ROWS * B_COLS

_info = plsc.get_sparse_core_info()
NC, NS = _info.num_cores, _info.num_subcores
NW = NC * NS

CHUNK = 128
PER_W = TOTAL // NW
N_CHUNKS = PER_W // CHUNK

_mesh = plsc.VectorSubcoreMesh(core_axis_name="c", subcore_axis_name="s")


@functools.partial(
    pl.kernel,
    mesh=_mesh,
    out_type=jax.ShapeDtypeStruct((TOTAL, EMBED_DIM), jnp.float32),
    scratch_types=[
        pltpu.VMEM((CHUNK,), jnp.int32),
        pltpu.VMEM((CHUNK, EMBED_DIM), jnp.float32),
        pltpu.SemaphoreType.DMA,
    ],
    compiler_params=pltpu.CompilerParams(use_tc_tiling_on_sc=False),
)
def _sc_gather(idx_hbm, table_hbm, out_hbm, idx_v, rows_v, sem):
    wid = lax.axis_index("s") * NC + lax.axis_index("c")
    base = wid * PER_W

    def body(i, carry):
        off = base + i * CHUNK
        pltpu.sync_copy(idx_hbm.at[pl.ds(off, CHUNK)], idx_v)
        pltpu.async_copy(table_hbm.at[idx_v], rows_v, sem).wait()
        pltpu.sync_copy(rows_v, out_hbm.at[pl.ds(off, CHUNK)])
        return carry

    lax.fori_loop(0, N_CHUNKS, body, 0)


def kernel(emb, table):
    idx = emb.reshape(-1)
    out = _sc_gather(idx, table)
    return out.reshape(B_ROWS, B_COLS, EMBED_DIM)


# speedup vs baseline: 4.8654x; 1.2935x over previous
import functools

import jax
import jax.numpy as jnp
from jax import lax
from jax.experimental import pallas as pl
from jax.experimental.pallas import tpu as pltpu
from jax.experimental.pallas import tpu_sc as plsc

NUM_EMBEDDINGS = 100000
EMBED_DIM = 64
B_ROWS = 4096
B_COLS = 100
TOTAL = B_ROWS * B_COLS

_info = plsc.get_sparse_core_info()
NC, NS = _info.num_cores, _info.num_subcores
NW = NC * NS

CHUNK = 128
NBUF = 5
PER_W = TOTAL // NW
N_CHUNKS = PER_W // CHUNK
N_GROUPS = N_CHUNKS // NBUF

_mesh = plsc.VectorSubcoreMesh(core_axis_name="c", subcore_axis_name="s")


@functools.partial(
    pl.kernel,
    mesh=_mesh,
    out_type=jax.ShapeDtypeStruct((TOTAL, EMBED_DIM), jnp.float32),
    scratch_types=[
        pltpu.VMEM((PER_W,), jnp.int32),
        pltpu.VMEM((NBUF, CHUNK, EMBED_DIM), jnp.float32),
        pltpu.SemaphoreType.DMA((NBUF,)),
        pltpu.SemaphoreType.DMA((NBUF,)),
    ],
    compiler_params=pltpu.CompilerParams(use_tc_tiling_on_sc=False),
)
def _sc_gather(idx_hbm, table_hbm, out_hbm, idx_v, rows_v, sem_g, sem_o):
    wid = lax.axis_index("s") * NC + lax.axis_index("c")
    base = wid * PER_W

    pltpu.sync_copy(idx_hbm.at[pl.ds(base, PER_W)], idx_v)

    def body(g, carry):
        goff = g * NBUF * CHUNK
        for b in range(NBUF):
            off = goff + b * CHUNK

            @pl.when(g > 0)
            def _wait_out():
                pltpu.make_async_copy(
                    rows_v.at[b], out_hbm.at[pl.ds(base, CHUNK)], sem_o.at[b]
                ).wait()

            pltpu.make_async_copy(
                table_hbm.at[idx_v.at[pl.ds(off, CHUNK)]],
                rows_v.at[b],
                sem_g.at[b],
            ).start()
        for b in range(NBUF):
            off = goff + b * CHUNK
            pltpu.make_async_copy(
                table_hbm.at[idx_v.at[pl.ds(off, CHUNK)]],
                rows_v.at[b],
                sem_g.at[b],
            ).wait()
            pltpu.make_async_copy(
                rows_v.at[b], out_hbm.at[pl.ds(base + off, CHUNK)], sem_o.at[b]
            ).start()
        return carry

    lax.fori_loop(0, N_GROUPS, body, 0)

    for b in range(NBUF):
        pltpu.make_async_copy(
            rows_v.at[b], out_hbm.at[pl.ds(base, CHUNK)], sem_o.at[b]
        ).wait()


def kernel(emb, table):
    idx = emb.reshape(-1)
    out = _sc_gather(idx, table)
    return out.reshape(B_ROWS, B_COLS, EMBED_DIM)


# speedup vs baseline: 4.8682x; 1.0006x over previous
import functools

import jax
import jax.numpy as jnp
from jax import lax
from jax.experimental import pallas as pl
from jax.experimental.pallas import tpu as pltpu
from jax.experimental.pallas import tpu_sc as plsc

NUM_EMBEDDINGS = 100000
EMBED_DIM = 64
B_ROWS = 4096
B_COLS = 100

_info = plsc.get_sparse_core_info()
NC, NS = _info.num_cores, _info.num_subcores
NW = NC * NS

NBUF = 8
ROWS_W = B_ROWS // NW
N_GROUPS = ROWS_W // NBUF

_mesh = plsc.VectorSubcoreMesh(core_axis_name="c", subcore_axis_name="s")


@functools.partial(
    pl.kernel,
    mesh=_mesh,
    out_type=jax.ShapeDtypeStruct((B_ROWS, B_COLS, EMBED_DIM), jnp.float32),
    scratch_types=[
        pltpu.VMEM((ROWS_W, B_COLS), jnp.int32),
        pltpu.VMEM((NBUF, B_COLS, EMBED_DIM), jnp.float32),
        pltpu.SemaphoreType.DMA((NBUF,)),
        pltpu.SemaphoreType.DMA((NBUF,)),
    ],
    compiler_params=pltpu.CompilerParams(use_tc_tiling_on_sc=False),
)
def _sc_gather(idx_hbm, table_hbm, out_hbm, idx_v, rows_v, sem_g, sem_o):
    wid = lax.axis_index("s") * NC + lax.axis_index("c")
    base = wid * ROWS_W

    pltpu.sync_copy(idx_hbm.at[pl.ds(base, ROWS_W)], idx_v)

    def body(g, carry):
        grow = g * NBUF
        for b in range(NBUF):

            @pl.when(g > 0)
            def _wait_out():
                pltpu.make_async_copy(
                    rows_v.at[b], out_hbm.at[base], sem_o.at[b]
                ).wait()

            pltpu.make_async_copy(
                table_hbm.at[idx_v.at[grow + b]],
                rows_v.at[b],
                sem_g.at[b],
            ).start()
        for b in range(NBUF):
            pltpu.make_async_copy(
                table_hbm.at[idx_v.at[grow + b]],
                rows_v.at[b],
                sem_g.at[b],
            ).wait()
            pltpu.make_async_copy(
                rows_v.at[b], out_hbm.at[base + grow + b], sem_o.at[b]
            ).start()
        return carry

    lax.fori_loop(0, N_GROUPS, body, 0)

    for b in range(NBUF):
        pltpu.make_async_copy(
            rows_v.at[b], out_hbm.at[base], sem_o.at[b]
        ).wait()


def kernel(emb, table):
    return _sc_gather(emb, table)
